# Initial kernel scaffold; baseline (speedup 1.0000x reference)
import jax
import jax.numpy as jnp
from jax.experimental import pallas as pl


def kernel(x, edge_index_intra, edge_index_inter, pos, batch, params):
    raise NotImplementedError("write your pallas kernel here")


# speedup vs baseline: 1.2787x; 1.2787x over previous
import functools

import jax
import jax.numpy as jnp
from jax import lax
from jax.experimental import pallas as pl
from jax.experimental.pallas import tpu as pltpu
from jax.experimental.pallas import tpu_sc as plsc

N = 10000
H = 256
HH = 128
E = 160000
G = 64
NC, NS = 2, 16
EPW = E // NS
EB = 80
NPAD = 10240
NPS = NPAD // NS
ZR = 128


def _edge_sc(h2, rad2, row, col):
    mesh = plsc.VectorSubcoreMesh(
        core_axis_name="c", subcore_axis_name="s",
        num_cores=NC, num_subcores=NS)

    @functools.partial(
        pl.kernel,
        out_type=jax.ShapeDtypeStruct((NC * NPAD, HH), jnp.float32),
        mesh=mesh,
        scratch_types=[
            pltpu.VMEM((EB,), jnp.int32),
            pltpu.VMEM((EB,), jnp.int32),
            pltpu.VMEM((EB,), jnp.int32),
            pltpu.VMEM((EB, HH), jnp.float32),
            pltpu.VMEM((EB, HH), jnp.float32),
            pltpu.VMEM((ZR, HH), jnp.float32),
            pltpu.VMEM_SHARED((NPAD, HH), jnp.float32),
            pltpu.SemaphoreType.DMA,
        ],
    )
    def k(h_hbm, rad_hbm, row_hbm, col_hbm, out_hbm,
          rowbuf, colbuf, gidx, hrows, radbuf, zbuf, acc, sem):
        c = lax.axis_index("c")
        s = lax.axis_index("s")
        zv = jnp.zeros((16,), jnp.float32)

        def zf(i, carry):
            r = i // (HH // 16)
            q = i % (HH // 16)
            zbuf[r, pl.ds(q * 16, 16)] = zv
            return carry
        lax.fori_loop(0, ZR * (HH // 16), zf, 0)
        for t in range(NPS // ZR):
            pltpu.sync_copy(zbuf, acc.at[pl.ds(s * NPS + t * ZR, ZR)])
        plsc.subcore_barrier()

        cN = c * N

        def body(kk, carry):
            base = s * EPW + kk * EB
            pltpu.sync_copy(row_hbm.at[pl.ds(base, EB)], rowbuf)
            pltpu.sync_copy(col_hbm.at[pl.ds(base, EB)], colbuf)
            for v in range(EB // 16):
                sl = pl.ds(v * 16, 16)
                gidx[sl] = rowbuf[sl] + cN
            pltpu.async_copy(h_hbm.at[gidx], hrows, sem).wait()
            pltpu.sync_copy(rad_hbm.at[pl.ds(c * E + base, EB)], radbuf)

            def mrow(j, carry2):
                for v in range(HH // 16):
                    sl = pl.ds(v * 16, 16)
                    hrows[j, sl] = hrows[j, sl] * radbuf[j, sl]
                return carry2
            lax.fori_loop(0, EB, mrow, 0)
            pltpu.sync_copy(hrows, acc.at[colbuf], add=True)
            return carry
        lax.fori_loop(0, EPW // EB, body, 0)

        plsc.subcore_barrier()
        pltpu.sync_copy(acc.at[pl.ds(s * NPS, NPS)],
                        out_hbm.at[pl.ds(c * NPAD + s * NPS, NPS)])

    return k(h2, rad2, row, col)


def _rbf(D):
    mu = jnp.linspace(0.0, 6.0, 9)
    sigma = 6.0 / 9
    return jnp.exp(-((D[..., None] - mu) / sigma) ** 2)


def _bn(h, g, b, eps=1e-5):
    m = h.mean(axis=0)
    v = h.var(axis=0)
    return g * (h - m) / jnp.sqrt(v + eps) + b


def _lrelu(x):
    return jnp.where(x >= 0, x, 0.01 * x)


def _silu(x):
    return x * jax.nn.sigmoid(x)


def _split2(a):
    return jnp.concatenate([a[:, :HH], a[:, HH:]], axis=0)


def _unsplit_pad(a2):
    return jnp.concatenate([a2[:N], a2[NPAD:NPAD + N]], axis=1)


def kernel(x, edge_index_intra, edge_index_inter, pos, batch, params):
    row_i = edge_index_intra[0].astype(jnp.int32)
    col_i = edge_index_intra[1].astype(jnp.int32)
    row_n = edge_index_inter[0].astype(jnp.int32)
    col_n = edge_index_inter[1].astype(jnp.int32)

    W, b = params["lin_node"]
    h = _silu(x @ W + b)

    def edge_d(rw, cl):
        diff = pos[rw] - pos[cl]
        return jnp.sqrt(jnp.sum(diff ** 2, axis=-1) + 1e-12)
    d_i = edge_d(row_i, col_i)
    d_n = edge_d(row_n, col_n)
    rbf_i = _rbf(d_i)
    rbf_n = _rbf(d_n)

    for p in params["hil"]:
        h2 = _split2(h)
        rad_i = _silu(rbf_i @ p["rc_lin"][0] + p["rc_lin"][1])
        rad_n = _silu(rbf_n @ p["rn_lin"][0] + p["rn_lin"][1])
        out_intra = _unsplit_pad(_edge_sc(h2, _split2(rad_i), row_i, col_i))
        out_inter = _unsplit_pad(_edge_sc(h2, _split2(rad_n), row_n, col_n))
        h1 = _bn(_lrelu((h + out_intra) @ p["cov_lin"][0] + p["cov_lin"][1]),
                 *p["cov_bn"])
        h2n = _bn(_lrelu((h + out_inter) @ p["ncov_lin"][0] + p["ncov_lin"][1]),
                  *p["ncov_bn"])
        h = h1 + h2n

    g = jax.ops.segment_sum(h, batch, num_segments=G)
    fc = params["fc"]
    for (Wl, bl), (ga, be) in zip(fc["lins"], fc["bns"]):
        g = _bn(_lrelu(g @ Wl + bl), ga, be)
    Wf, bf = fc["final"]
    return (g @ Wf + bf).reshape(-1)


# speedup vs baseline: 1.4246x; 1.1141x over previous
import functools

import jax
import jax.numpy as jnp
from jax import lax
from jax.experimental import pallas as pl
from jax.experimental.pallas import tpu as pltpu
from jax.experimental.pallas import tpu_sc as plsc

N = 10000
H = 256
HH = 128
E = 160000
G = 64
NC, NS = 2, 16
EPW = E // NS
EB = 80
NPS = 624
ZR = 128

EPAD = 327680
RB = 128
RCH = EPAD // (128 * NC * NS)

def _sc_d2(posp, rowp, colp):
    mesh = plsc.VectorSubcoreMesh(
        core_axis_name="c", subcore_axis_name="s",
        num_cores=NC, num_subcores=NS)

    @functools.partial(
        pl.kernel,
        out_type=jax.ShapeDtypeStruct((EPAD * 16,), jnp.float32),
        mesh=mesh,
        scratch_types=[
            pltpu.VMEM((RB,), jnp.int32),
            pltpu.VMEM((RB,), jnp.int32),
            pltpu.VMEM((RB, 128), jnp.float32),
            pltpu.VMEM((RB, 128), jnp.float32),
            pltpu.VMEM((RB * 16,), jnp.float32),
            pltpu.SemaphoreType.DMA,
        ],
    )
    def k(pos_hbm, row_hbm, col_hbm, out_hbm,
          rowbuf, colbuf, prbuf, pcbuf, obuf, sem):
        c = lax.axis_index("c")
        s = lax.axis_index("s")
        wid = c * NS + s

        def chunk(kk, carry):
            base = (wid * RCH + kk) * RB
            pltpu.sync_copy(row_hbm.at[pl.ds(base, RB)], rowbuf)
            pltpu.sync_copy(col_hbm.at[pl.ds(base, RB)], colbuf)
            pltpu.async_copy(pos_hbm.at[rowbuf], prbuf, sem).wait()
            pltpu.async_copy(pos_hbm.at[colbuf], pcbuf, sem).wait()

            def edge(j, carry2):
                sl = pl.ds(0, 16)
                diff = prbuf[j, sl] - pcbuf[j, sl]
                sq = diff * diff
                d2s = sq[0] + sq[1] + sq[2]
                obuf[pl.ds(j * 16, 16)] = d2s + jnp.zeros((16,), jnp.float32)
                return carry2
            lax.fori_loop(0, RB, edge, 0)
            pltpu.sync_copy(obuf, out_hbm.at[pl.ds(base * 16, RB * 16)])
            return carry
        lax.fori_loop(0, RCH, chunk, 0)

    return k(posp, rowp, colp)


def _edge_sc(h2, rad2, row, col):
    mesh = plsc.VectorSubcoreMesh(
        core_axis_name="c", subcore_axis_name="s",
        num_cores=NC, num_subcores=NS)

    @functools.partial(
        pl.kernel,
        out_type=jax.ShapeDtypeStruct((NC * N, HH), jnp.float32),
        mesh=mesh,
        scratch_types=[
            pltpu.VMEM((EB,), jnp.int32),
            pltpu.VMEM((EB,), jnp.int32),
            pltpu.VMEM((EB,), jnp.int32),
            pltpu.VMEM((EB, HH), jnp.float32),
            pltpu.VMEM((EB, HH), jnp.float32),
            pltpu.VMEM((ZR, HH), jnp.float32),
            pltpu.VMEM_SHARED((N, HH), jnp.float32),
            pltpu.SemaphoreType.DMA,
        ],
    )
    def k(h_hbm, rad_hbm, row_hbm, col_hbm, out_hbm,
          rowbuf, colbuf, gidx, hrows, radbuf, zbuf, acc, sem):
        c = lax.axis_index("c")
        s = lax.axis_index("s")
        zv = jnp.zeros((16,), jnp.float32)

        def zf(i, carry):
            r = i // (HH // 16)
            q = i % (HH // 16)
            zbuf[r, pl.ds(q * 16, 16)] = zv
            return carry
        lax.fori_loop(0, ZR * (HH // 16), zf, 0)
        for t in range(5):
            pltpu.sync_copy(zbuf, acc.at[pl.ds(s * NPS + t * ZR, ZR)])
        plsc.subcore_barrier()

        cN = c * N

        def body(kk, carry):
            base = s * EPW + kk * EB
            pltpu.sync_copy(row_hbm.at[pl.ds(base, EB)], rowbuf)
            pltpu.sync_copy(col_hbm.at[pl.ds(base, EB)], colbuf)
            for v in range(EB // 16):
                sl = pl.ds(v * 16, 16)
                gidx[sl] = rowbuf[sl] + cN
            pltpu.async_copy(h_hbm.at[gidx], hrows, sem).wait()
            pltpu.sync_copy(rad_hbm.at[pl.ds(c * E + base, EB)], radbuf)

            def mrow(j, carry2):
                for v in range(HH // 16):
                    sl = pl.ds(v * 16, 16)
                    hrows[j, sl] = hrows[j, sl] * radbuf[j, sl]
                return carry2
            lax.fori_loop(0, EB, mrow, 0)
            pltpu.sync_copy(hrows, acc.at[colbuf], add=True)
            return carry
        lax.fori_loop(0, EPW // EB, body, 0)

        plsc.subcore_barrier()
        pltpu.sync_copy(acc.at[pl.ds(s * NPS, 640)],
                        out_hbm.at[pl.ds(cN + s * NPS, 640)])

    return k(h2, rad2, row, col)


BN = 1000
NBLK = N // BN
BE = 1280
NEBLK = E // BE


def _silu(x):
    return x * jax.nn.sigmoid(x)


def _lrelu(x):
    return jnp.where(x >= 0, x, 0.01 * x)


def _lin0(x, W, b):
    def body(x_ref, w_ref, b_ref, o_ref):
        y = _silu(jnp.dot(x_ref[...], w_ref[...],
                          preferred_element_type=jnp.float32) + b_ref[...])
        o_ref[0] = y[:, :HH]
        o_ref[1] = y[:, HH:]

    out = pl.pallas_call(
        body,
        grid=(NBLK,),
        in_specs=[
            pl.BlockSpec((BN, H), lambda i: (i, 0)),
            pl.BlockSpec((H, H), lambda i: (0, 0)),
            pl.BlockSpec((1, H), lambda i: (0, 0)),
        ],
        out_specs=pl.BlockSpec((2, BN, HH), lambda i: (0, i, 0)),
        out_shape=jax.ShapeDtypeStruct((2, N, HH), jnp.float32),
    )(x, W, b.reshape(1, H))
    return out.reshape(2 * N, HH)


def _radial(d2E, W, b, off):
    def body(r_ref, w_ref, b_ref, o_ref):
        d = jnp.sqrt(r_ref[...] + 1e-12)
        mu = lax.broadcasted_iota(jnp.int32, (BE, 16), 1).astype(jnp.float32) * 0.75
        t = (d - mu) * 1.5
        rbf = jnp.exp(-(t * t))
        y = jnp.dot(rbf, w_ref[...], preferred_element_type=jnp.float32)
        y = _silu(y + b_ref[...])
        o_ref[0] = y[:, :HH]
        o_ref[1] = y[:, HH:]

    out = pl.pallas_call(
        body,
        grid=(NEBLK,),
        in_specs=[
            pl.BlockSpec((BE, 16), lambda i: (i + off, 0)),
            pl.BlockSpec((16, H), lambda i: (0, 0)),
            pl.BlockSpec((1, H), lambda i: (0, 0)),
        ],
        out_specs=pl.BlockSpec((2, BE, HH), lambda i: (0, i, 0)),
        out_shape=jax.ShapeDtypeStruct((2, E, HH), jnp.float32),
    )(d2E, W, b.reshape(1, H))
    return out.reshape(2 * E, HH)


def _cov(h2, oi2, on2, W1, b1, W2, b2):
    def body(hl, hh, oil, oih, onl, onh, w1, bb1, w2, bb2,
             y1_ref, y2_ref, st_ref):
        i = pl.program_id(0)
        al = hl[...] + oil[...]
        ah = hh[...] + oih[...]
        y1 = _lrelu(jnp.dot(al, w1[:HH, :], preferred_element_type=jnp.float32)
                    + jnp.dot(ah, w1[HH:, :], preferred_element_type=jnp.float32)
                    + bb1[...])
        bl = hl[...] + onl[...]
        bh = hh[...] + onh[...]
        y2 = _lrelu(jnp.dot(bl, w2[:HH, :], preferred_element_type=jnp.float32)
                    + jnp.dot(bh, w2[HH:, :], preferred_element_type=jnp.float32)
                    + bb2[...])
        y1_ref[0] = y1[:, :HH]
        y1_ref[1] = y1[:, HH:]
        y2_ref[0] = y2[:, :HH]
        y2_ref[1] = y2[:, HH:]

        @pl.when(i == 0)
        def _():
            st_ref[...] = jnp.zeros_like(st_ref)
        st = jnp.concatenate([
            jnp.sum(y1, axis=0, keepdims=True),
            jnp.sum(y1 * y1, axis=0, keepdims=True),
            jnp.sum(y2, axis=0, keepdims=True),
            jnp.sum(y2 * y2, axis=0, keepdims=True),
        ], axis=0)
        st_ref[0:4, :] = st_ref[0:4, :] + st

    lo = lambda i: (i, 0)
    hi = lambda i: (i + NBLK, 0)
    cst = lambda i: (0, 0)
    y1, y2, stats = pl.pallas_call(
        body,
        grid=(NBLK,),
        in_specs=[
            pl.BlockSpec((BN, HH), lo), pl.BlockSpec((BN, HH), hi),
            pl.BlockSpec((BN, HH), lo), pl.BlockSpec((BN, HH), hi),
            pl.BlockSpec((BN, HH), lo), pl.BlockSpec((BN, HH), hi),
            pl.BlockSpec((H, H), cst), pl.BlockSpec((1, H), cst),
            pl.BlockSpec((H, H), cst), pl.BlockSpec((1, H), cst),
        ],
        out_specs=[
            pl.BlockSpec((2, BN, HH), lambda i: (0, i, 0)),
            pl.BlockSpec((2, BN, HH), lambda i: (0, i, 0)),
            pl.BlockSpec((8, H), cst),
        ],
        out_shape=[
            jax.ShapeDtypeStruct((2, N, HH), jnp.float32),
            jax.ShapeDtypeStruct((2, N, HH), jnp.float32),
            jax.ShapeDtypeStruct((8, H), jnp.float32),
        ],
    )(h2, h2, oi2, oi2, on2, on2, W1, b1.reshape(1, H), W2, b2.reshape(1, H))
    return y1.reshape(2 * N, HH), y2.reshape(2 * N, HH), stats


def _comb(y1, y2, stats, g1, be1, g2, be2, eps=1e-5):
    def body(y1l, y1h, y2l, y2h, st, gg1, bb1, gg2, bb2, o_ref):
        s = st[...]
        m1 = s[0:1, :] / N
        v1 = s[1:2, :] / N - m1 * m1
        sc1 = gg1[...] * lax.rsqrt(v1 + eps)
        sh1 = bb1[...] - m1 * sc1
        m2 = s[2:3, :] / N
        v2 = s[3:4, :] / N - m2 * m2
        sc2 = gg2[...] * lax.rsqrt(v2 + eps)
        sh2 = bb2[...] - m2 * sc2
        o_ref[0] = (y1l[...] * sc1[:, :HH] + sh1[:, :HH]
                    + y2l[...] * sc2[:, :HH] + sh2[:, :HH])
        o_ref[1] = (y1h[...] * sc1[:, HH:] + sh1[:, HH:]
                    + y2h[...] * sc2[:, HH:] + sh2[:, HH:])

    lo = lambda i: (i, 0)
    hi = lambda i: (i + NBLK, 0)
    cst = lambda i: (0, 0)
    out = pl.pallas_call(
        body,
        grid=(NBLK,),
        in_specs=[
            pl.BlockSpec((BN, HH), lo), pl.BlockSpec((BN, HH), hi),
            pl.BlockSpec((BN, HH), lo), pl.BlockSpec((BN, HH), hi),
            pl.BlockSpec((8, H), cst),
            pl.BlockSpec((1, H), cst), pl.BlockSpec((1, H), cst),
            pl.BlockSpec((1, H), cst), pl.BlockSpec((1, H), cst),
        ],
        out_specs=pl.BlockSpec((2, BN, HH), lambda i: (0, i, 0)),
        out_shape=jax.ShapeDtypeStruct((2, N, HH), jnp.float32),
    )(y1, y1, y2, y2, stats,
      g1.reshape(1, H), be1.reshape(1, H), g2.reshape(1, H), be2.reshape(1, H))
    return out.reshape(2 * N, HH)


def _seg(h2, batch3):
    def body(hl, hh, b_ref, g_ref):
        i = pl.program_id(0)

        @pl.when(i == 0)
        def _():
            g_ref[...] = jnp.zeros_like(g_ref)
        bb = b_ref[0]
        seg = lax.broadcasted_iota(jnp.int32, (G, BN), 0)
        oh = jnp.where(seg == jnp.broadcast_to(bb, (G, BN)), 1.0, 0.0)
        g_ref[:, :HH] = g_ref[:, :HH] + jnp.dot(
            oh, hl[...], preferred_element_type=jnp.float32)
        g_ref[:, HH:] = g_ref[:, HH:] + jnp.dot(
            oh, hh[...], preferred_element_type=jnp.float32)

    lo = lambda i: (i, 0)
    hi = lambda i: (i + NBLK, 0)
    return pl.pallas_call(
        body,
        grid=(NBLK,),
        in_specs=[
            pl.BlockSpec((BN, HH), lo), pl.BlockSpec((BN, HH), hi),
            pl.BlockSpec((1, 1, BN), lambda i: (i, 0, 0)),
        ],
        out_specs=pl.BlockSpec((G, H), lambda i: (0, 0)),
        out_shape=jax.ShapeDtypeStruct((G, H), jnp.float32),
    )(h2, h2, batch3)


def _head(g, fc, eps=1e-5):
    Ws = [w for w, _ in fc["lins"]]
    bs = [b for _, b in fc["lins"]]
    gs = [ga for ga, _ in fc["bns"]]
    es = [be for _, be in fc["bns"]]
    Wf, bf = fc["final"]

    def body(g_ref, w0, b0, g0, e0, w1, b1, g1, e1, w2, b2, g2, e2,
             wf, bfr, o_ref):
        gg = g_ref[...]
        for w, b, ga, be in ((w0, b0, g0, e0), (w1, b1, g1, e1),
                             (w2, b2, g2, e2)):
            y = _lrelu(jnp.dot(gg, w[...],
                               preferred_element_type=jnp.float32) + b[...])
            m = jnp.mean(y, axis=0, keepdims=True)
            v = jnp.mean(y * y, axis=0, keepdims=True) - m * m
            gg = ga[...] * (y - m) * lax.rsqrt(v + eps) + be[...]
        res = jnp.dot(gg, wf[...], preferred_element_type=jnp.float32) + bfr[...]
        o_ref[...] = jnp.broadcast_to(res, (G, HH))

    cst = lambda: (0, 0)
    args = [g]
    in_specs = [pl.BlockSpec((G, H), cst)]
    for w, b, ga, be in zip(Ws, bs, gs, es):
        args += [w, b.reshape(1, H), ga.reshape(1, H), be.reshape(1, H)]
        in_specs += [pl.BlockSpec((H, H), cst), pl.BlockSpec((1, H), cst),
                     pl.BlockSpec((1, H), cst), pl.BlockSpec((1, H), cst)]
    args += [Wf, bf.reshape(1, 1)]
    in_specs += [pl.BlockSpec((H, 1), cst), pl.BlockSpec((1, 1), cst)]
    out = pl.pallas_call(
        body,
        in_specs=in_specs,
        out_specs=pl.BlockSpec((G, HH), cst),
        out_shape=jax.ShapeDtypeStruct((G, HH), jnp.float32),
    )(*args)
    return out[:, 0]


def kernel(x, edge_index_intra, edge_index_inter, pos, batch, params):
    row_i = edge_index_intra[0].astype(jnp.int32)
    col_i = edge_index_intra[1].astype(jnp.int32)
    row_n = edge_index_inter[0].astype(jnp.int32)
    col_n = edge_index_inter[1].astype(jnp.int32)

    posp = jnp.pad(pos.astype(jnp.float32), ((0, 0), (0, 125)))
    rowp = jnp.pad(jnp.concatenate([row_i, row_n]), (0, EPAD - 2 * E))
    colp = jnp.pad(jnp.concatenate([col_i, col_n]), (0, EPAD - 2 * E))
    d2E = _sc_d2(posp, rowp, colp).reshape(EPAD, 16)

    W, b = params["lin_node"]
    h2 = _lin0(x, W, b)

    batch3 = batch.astype(jnp.int32).reshape(NBLK, 1, BN)

    for p in params["hil"]:
        rcW = jnp.pad(p["rc_lin"][0], ((0, 7), (0, 0)))
        rnW = jnp.pad(p["rn_lin"][0], ((0, 7), (0, 0)))
        rad_i = _radial(d2E, rcW, p["rc_lin"][1], 0)
        rad_n = _radial(d2E, rnW, p["rn_lin"][1], NEBLK)
        oi2 = _edge_sc(h2, rad_i, row_i, col_i)
        on2 = _edge_sc(h2, rad_n, row_n, col_n)
        y1, y2, stats = _cov(h2, oi2, on2,
                             p["cov_lin"][0], p["cov_lin"][1],
                             p["ncov_lin"][0], p["ncov_lin"][1])
        h2 = _comb(y1, y2, stats,
                   p["cov_bn"][0], p["cov_bn"][1],
                   p["ncov_bn"][0], p["ncov_bn"][1])

    g = _seg(h2, batch3)
    return _head(g, params["fc"])


# speedup vs baseline: 1.5945x; 1.1193x over previous
import functools

import jax
import jax.numpy as jnp
from jax import lax
from jax.experimental import pallas as pl
from jax.experimental.pallas import tpu as pltpu
from jax.experimental.pallas import tpu_sc as plsc

N = 10000
H = 256
HH = 128
E = 160000
G = 64
NC, NS = 2, 16
EPW = E // NS
EB = 80
NPS = 624
ZR = 128

EPAD = 327680
RB = 128
RCH = EPAD // (128 * NC * NS)

def _sc_d2(posp, rowp, colp):
    mesh = plsc.VectorSubcoreMesh(
        core_axis_name="c", subcore_axis_name="s",
        num_cores=NC, num_subcores=NS)

    @functools.partial(
        pl.kernel,
        out_type=jax.ShapeDtypeStruct((EPAD * 16,), jnp.float32),
        mesh=mesh,
        scratch_types=[
            pltpu.VMEM((RB,), jnp.int32), pltpu.VMEM((RB,), jnp.int32),
            pltpu.VMEM((RB,), jnp.int32), pltpu.VMEM((RB,), jnp.int32),
            pltpu.VMEM((RB, 128), jnp.float32), pltpu.VMEM((RB, 128), jnp.float32),
            pltpu.VMEM((RB, 128), jnp.float32), pltpu.VMEM((RB, 128), jnp.float32),
            pltpu.VMEM((RB * 16,), jnp.float32), pltpu.VMEM((RB * 16,), jnp.float32),
            pltpu.SemaphoreType.DMA, pltpu.SemaphoreType.DMA,
            pltpu.SemaphoreType.DMA, pltpu.SemaphoreType.DMA,
        ],
    )
    def k(pos_hbm, row_hbm, col_hbm, out_hbm,
          rw0, rw1, cl0, cl1, pr0, pr1, pc0, pc1, ob0, ob1,
          si0, si1, sp0, sp1):
        c = lax.axis_index("c")
        s = lax.axis_index("s")
        wid = c * NS + s
        BUF = ((rw0, cl0, pr0, pc0, ob0, si0, sp0),
               (rw1, cl1, pr1, pc1, ob1, si1, sp1))

        def base_of(g):
            return (wid * RCH + jnp.minimum(g, RCH - 1)) * RB

        def issue_in(g, b):
            rw, cl, pr, pc, ob, si, sp = BUF[b]
            base = base_of(g)
            pltpu.async_copy(row_hbm.at[pl.ds(base, RB)], rw, si)
            pltpu.async_copy(col_hbm.at[pl.ds(base, RB)], cl, si)

        def wait_in(b):
            rw, cl, pr, pc, ob, si, sp = BUF[b]
            pltpu.make_async_copy(row_hbm.at[pl.ds(0, RB)], rw, si).wait()
            pltpu.make_async_copy(col_hbm.at[pl.ds(0, RB)], cl, si).wait()

        def issue_gather(b):
            rw, cl, pr, pc, ob, si, sp = BUF[b]
            wait_in(b)
            pltpu.async_copy(pos_hbm.at[rw], pr, sp)
            pltpu.async_copy(pos_hbm.at[cl], pc, sp)

        def wait_gather(b):
            rw, cl, pr, pc, ob, si, sp = BUF[b]
            pltpu.make_async_copy(pos_hbm.at[rw], pr, sp).wait()
            pltpu.make_async_copy(pos_hbm.at[cl], pc, sp).wait()

        def compute(g, b):
            rw, cl, pr, pc, ob, si, sp = BUF[b]
            wait_gather(b)

            def edge(j, carry2):
                sl = pl.ds(0, 16)
                diff = pr[j, sl] - pc[j, sl]
                ob[pl.ds(j * 16, 16)] = diff * diff
                return carry2
            lax.fori_loop(0, RB, edge, 0, unroll=4)
            pltpu.sync_copy(ob, out_hbm.at[pl.ds(base_of(g) * 16, RB * 16)])

        issue_in(0, 0)
        issue_in(1, 1)
        issue_gather(0)

        def body(p, carry):
            g = 2 * p
            issue_gather(1)
            compute(g, 0)
            issue_in(g + 2, 0)
            issue_gather(0)
            compute(g + 1, 1)
            issue_in(g + 3, 1)
            return carry
        lax.fori_loop(0, RCH // 2, body, 0)
        wait_gather(0)
        wait_in(1)

    return k(posp, rowp, colp)


def _edge_sc(h2, rad2, row, col):
    mesh = plsc.VectorSubcoreMesh(
        core_axis_name="c", subcore_axis_name="s",
        num_cores=NC, num_subcores=NS)

    NCH = EPW // EB

    @functools.partial(
        pl.kernel,
        out_type=jax.ShapeDtypeStruct((NC * N, HH), jnp.float32),
        mesh=mesh,
        scratch_types=[
            pltpu.VMEM((EB,), jnp.int32), pltpu.VMEM((EB,), jnp.int32),
            pltpu.VMEM((EB,), jnp.int32), pltpu.VMEM((EB,), jnp.int32),
            pltpu.VMEM((EB,), jnp.int32), pltpu.VMEM((EB,), jnp.int32),
            pltpu.VMEM((EB, HH), jnp.float32), pltpu.VMEM((EB, HH), jnp.float32),
            pltpu.VMEM((EB, HH), jnp.float32), pltpu.VMEM((EB, HH), jnp.float32),
            pltpu.VMEM_SHARED((N, HH), jnp.float32),
            pltpu.SemaphoreType.DMA, pltpu.SemaphoreType.DMA,
            pltpu.SemaphoreType.DMA, pltpu.SemaphoreType.DMA,
            pltpu.SemaphoreType.DMA, pltpu.SemaphoreType.DMA,
        ],
    )
    def k(h_hbm, rad_hbm, row_hbm, col_hbm, out_hbm,
          rw0, rw1, cl0, cl1, gx0, gx1, hr0, hr1, rd0, rd1,
          acc, si0, si1, sr0, sr1, sg0, sg1):
        c = lax.axis_index("c")
        s = lax.axis_index("s")
        BUF = ((rw0, cl0, gx0, hr0, rd0, si0, sr0, sg0),
               (rw1, cl1, gx1, hr1, rd1, si1, sr1, sg1))
        zv = jnp.zeros((16,), jnp.float32)

        def zf(i, carry):
            r = i // (HH // 16)
            q = i % (HH // 16)
            hr0[r, pl.ds(q * 16, 16)] = zv
            return carry
        lax.fori_loop(0, EB * (HH // 16), zf, 0)
        for t in range(8):
            pltpu.sync_copy(hr0, acc.at[pl.ds(s * NPS + t * EB, EB)])
        plsc.subcore_barrier()

        cN = c * N

        def base_of(g):
            return s * EPW + jnp.minimum(g, NCH - 1) * EB

        def issue_in(g, b):
            rw, cl, gx, hr, rd, si, sr, sg = BUF[b]
            base = base_of(g)
            pltpu.async_copy(row_hbm.at[pl.ds(base, EB)], rw, si)
            pltpu.async_copy(col_hbm.at[pl.ds(base, EB)], cl, si)
            pltpu.async_copy(rad_hbm.at[pl.ds(c * E + base, EB)], rd, sr)

        def wait_in(b):
            rw, cl, gx, hr, rd, si, sr, sg = BUF[b]
            pltpu.make_async_copy(row_hbm.at[pl.ds(0, EB)], rw, si).wait()
            pltpu.make_async_copy(col_hbm.at[pl.ds(0, EB)], cl, si).wait()

        def issue_gather(b):
            rw, cl, gx, hr, rd, si, sr, sg = BUF[b]
            wait_in(b)
            for v in range(EB // 16):
                sl = pl.ds(v * 16, 16)
                gx[sl] = rw[sl] + cN
            pltpu.async_copy(h_hbm.at[gx], hr, sg)

        def mul_scatter(b):
            rw, cl, gx, hr, rd, si, sr, sg = BUF[b]
            pltpu.make_async_copy(h_hbm.at[gx], hr, sg).wait()
            pltpu.make_async_copy(rad_hbm.at[pl.ds(0, EB)], rd, sr).wait()

            def mrow(j, carry2):
                for v in range(HH // 16):
                    sl = pl.ds(v * 16, 16)
                    hr[j, sl] = hr[j, sl] * rd[j, sl]
                return carry2
            lax.fori_loop(0, EB, mrow, 0, unroll=2)
            pltpu.sync_copy(hr, acc.at[cl], add=True)

        issue_in(0, 0)
        issue_in(1, 1)
        issue_gather(0)

        def body(p, carry):
            g = 2 * p
            issue_gather(1)
            mul_scatter(0)
            issue_in(g + 2, 0)
            issue_gather(0)
            mul_scatter(1)
            issue_in(g + 3, 1)
            return carry
        lax.fori_loop(0, (NCH - 1) // 2, body, 0)
        mul_scatter(0)
        wait_in(1)
        rw, cl, gx, hr, rd, si, sr, sg = BUF[1]
        pltpu.make_async_copy(rad_hbm.at[pl.ds(0, EB)], rd, sr).wait()

        plsc.subcore_barrier()
        pltpu.sync_copy(acc.at[pl.ds(s * NPS, 640)],
                        out_hbm.at[pl.ds(cN + s * NPS, 640)])

    return k(h2, rad2, row, col)


BN = 1000
NBLK = N // BN
BE = 1280
NEBLK = E // BE


def _silu(x):
    return x * jax.nn.sigmoid(x)


def _lrelu(x):
    return jnp.where(x >= 0, x, 0.01 * x)


def _lin0(x, W, b):
    def body(x_ref, w_ref, b_ref, o_ref):
        y = _silu(jnp.dot(x_ref[...], w_ref[...],
                          preferred_element_type=jnp.float32) + b_ref[...])
        o_ref[0] = y[:, :HH]
        o_ref[1] = y[:, HH:]

    out = pl.pallas_call(
        body,
        grid=(NBLK,),
        in_specs=[
            pl.BlockSpec((BN, H), lambda i: (i, 0)),
            pl.BlockSpec((H, H), lambda i: (0, 0)),
            pl.BlockSpec((1, H), lambda i: (0, 0)),
        ],
        out_specs=pl.BlockSpec((2, BN, HH), lambda i: (0, i, 0)),
        out_shape=jax.ShapeDtypeStruct((2, N, HH), jnp.float32),
    )(x, W, b.reshape(1, H))
    return out.reshape(2 * N, HH)


def _radial(d2E, W, b, off):
    def body(r_ref, w_ref, b_ref, o_ref):
        sq = r_ref[...]
        d2 = sq[:, 0:1] + sq[:, 1:2] + sq[:, 2:3]
        d = jnp.sqrt(d2 + 1e-12)
        mu = lax.broadcasted_iota(jnp.int32, (BE, 16), 1).astype(jnp.float32) * 0.75
        t = (d - mu) * 1.5
        rbf = jnp.exp(-(t * t))
        y = jnp.dot(rbf, w_ref[...], preferred_element_type=jnp.float32)
        y = _silu(y + b_ref[...])
        o_ref[0] = y[:, :HH]
        o_ref[1] = y[:, HH:]

    out = pl.pallas_call(
        body,
        grid=(NEBLK,),
        in_specs=[
            pl.BlockSpec((BE, 16), lambda i: (i + off, 0)),
            pl.BlockSpec((16, H), lambda i: (0, 0)),
            pl.BlockSpec((1, H), lambda i: (0, 0)),
        ],
        out_specs=pl.BlockSpec((2, BE, HH), lambda i: (0, i, 0)),
        out_shape=jax.ShapeDtypeStruct((2, E, HH), jnp.float32),
    )(d2E, W, b.reshape(1, H))
    return out.reshape(2 * E, HH)


def _cov(h2, oi2, on2, W1, b1, W2, b2):
    def body(hl, hh, oil, oih, onl, onh, w1, bb1, w2, bb2,
             y1_ref, y2_ref, st_ref):
        i = pl.program_id(0)
        al = hl[...] + oil[...]
        ah = hh[...] + oih[...]
        y1 = _lrelu(jnp.dot(al, w1[:HH, :], preferred_element_type=jnp.float32)
                    + jnp.dot(ah, w1[HH:, :], preferred_element_type=jnp.float32)
                    + bb1[...])
        bl = hl[...] + onl[...]
        bh = hh[...] + onh[...]
        y2 = _lrelu(jnp.dot(bl, w2[:HH, :], preferred_element_type=jnp.float32)
                    + jnp.dot(bh, w2[HH:, :], preferred_element_type=jnp.float32)
                    + bb2[...])
        y1_ref[0] = y1[:, :HH]
        y1_ref[1] = y1[:, HH:]
        y2_ref[0] = y2[:, :HH]
        y2_ref[1] = y2[:, HH:]

        @pl.when(i == 0)
        def _():
            st_ref[...] = jnp.zeros_like(st_ref)
        st = jnp.concatenate([
            jnp.sum(y1, axis=0, keepdims=True),
            jnp.sum(y1 * y1, axis=0, keepdims=True),
            jnp.sum(y2, axis=0, keepdims=True),
            jnp.sum(y2 * y2, axis=0, keepdims=True),
        ], axis=0)
        st_ref[0:4, :] = st_ref[0:4, :] + st

    lo = lambda i: (i, 0)
    hi = lambda i: (i + NBLK, 0)
    cst = lambda i: (0, 0)
    y1, y2, stats = pl.pallas_call(
        body,
        grid=(NBLK,),
        in_specs=[
            pl.BlockSpec((BN, HH), lo), pl.BlockSpec((BN, HH), hi),
            pl.BlockSpec((BN, HH), lo), pl.BlockSpec((BN, HH), hi),
            pl.BlockSpec((BN, HH), lo), pl.BlockSpec((BN, HH), hi),
            pl.BlockSpec((H, H), cst), pl.BlockSpec((1, H), cst),
            pl.BlockSpec((H, H), cst), pl.BlockSpec((1, H), cst),
        ],
        out_specs=[
            pl.BlockSpec((2, BN, HH), lambda i: (0, i, 0)),
            pl.BlockSpec((2, BN, HH), lambda i: (0, i, 0)),
            pl.BlockSpec((8, H), cst),
        ],
        out_shape=[
            jax.ShapeDtypeStruct((2, N, HH), jnp.float32),
            jax.ShapeDtypeStruct((2, N, HH), jnp.float32),
            jax.ShapeDtypeStruct((8, H), jnp.float32),
        ],
    )(h2, h2, oi2, oi2, on2, on2, W1, b1.reshape(1, H), W2, b2.reshape(1, H))
    return y1.reshape(2 * N, HH), y2.reshape(2 * N, HH), stats


def _comb(y1, y2, stats, g1, be1, g2, be2, eps=1e-5):
    def body(y1l, y1h, y2l, y2h, st, gg1, bb1, gg2, bb2, o_ref):
        s = st[...]
        m1 = s[0:1, :] / N
        v1 = s[1:2, :] / N - m1 * m1
        sc1 = gg1[...] * lax.rsqrt(v1 + eps)
        sh1 = bb1[...] - m1 * sc1
        m2 = s[2:3, :] / N
        v2 = s[3:4, :] / N - m2 * m2
        sc2 = gg2[...] * lax.rsqrt(v2 + eps)
        sh2 = bb2[...] - m2 * sc2
        o_ref[0] = (y1l[...] * sc1[:, :HH] + sh1[:, :HH]
                    + y2l[...] * sc2[:, :HH] + sh2[:, :HH])
        o_ref[1] = (y1h[...] * sc1[:, HH:] + sh1[:, HH:]
                    + y2h[...] * sc2[:, HH:] + sh2[:, HH:])

    lo = lambda i: (i, 0)
    hi = lambda i: (i + NBLK, 0)
    cst = lambda i: (0, 0)
    out = pl.pallas_call(
        body,
        grid=(NBLK,),
        in_specs=[
            pl.BlockSpec((BN, HH), lo), pl.BlockSpec((BN, HH), hi),
            pl.BlockSpec((BN, HH), lo), pl.BlockSpec((BN, HH), hi),
            pl.BlockSpec((8, H), cst),
            pl.BlockSpec((1, H), cst), pl.BlockSpec((1, H), cst),
            pl.BlockSpec((1, H), cst), pl.BlockSpec((1, H), cst),
        ],
        out_specs=pl.BlockSpec((2, BN, HH), lambda i: (0, i, 0)),
        out_shape=jax.ShapeDtypeStruct((2, N, HH), jnp.float32),
    )(y1, y1, y2, y2, stats,
      g1.reshape(1, H), be1.reshape(1, H), g2.reshape(1, H), be2.reshape(1, H))
    return out.reshape(2 * N, HH)


def _seg(h2, batch3):
    def body(hl, hh, b_ref, g_ref):
        i = pl.program_id(0)

        @pl.when(i == 0)
        def _():
            g_ref[...] = jnp.zeros_like(g_ref)
        bb = b_ref[0]
        seg = lax.broadcasted_iota(jnp.int32, (G, BN), 0)
        oh = jnp.where(seg == jnp.broadcast_to(bb, (G, BN)), 1.0, 0.0)
        g_ref[:, :HH] = g_ref[:, :HH] + jnp.dot(
            oh, hl[...], preferred_element_type=jnp.float32)
        g_ref[:, HH:] = g_ref[:, HH:] + jnp.dot(
            oh, hh[...], preferred_element_type=jnp.float32)

    lo = lambda i: (i, 0)
    hi = lambda i: (i + NBLK, 0)
    return pl.pallas_call(
        body,
        grid=(NBLK,),
        in_specs=[
            pl.BlockSpec((BN, HH), lo), pl.BlockSpec((BN, HH), hi),
            pl.BlockSpec((1, 1, BN), lambda i: (i, 0, 0)),
        ],
        out_specs=pl.BlockSpec((G, H), lambda i: (0, 0)),
        out_shape=jax.ShapeDtypeStruct((G, H), jnp.float32),
    )(h2, h2, batch3)


def _head(g, fc, eps=1e-5):
    Ws = [w for w, _ in fc["lins"]]
    bs = [b for _, b in fc["lins"]]
    gs = [ga for ga, _ in fc["bns"]]
    es = [be for _, be in fc["bns"]]
    Wf, bf = fc["final"]

    def body(g_ref, w0, b0, g0, e0, w1, b1, g1, e1, w2, b2, g2, e2,
             wf, bfr, o_ref):
        gg = g_ref[...]
        for w, b, ga, be in ((w0, b0, g0, e0), (w1, b1, g1, e1),
                             (w2, b2, g2, e2)):
            y = _lrelu(jnp.dot(gg, w[...],
                               preferred_element_type=jnp.float32) + b[...])
            m = jnp.mean(y, axis=0, keepdims=True)
            v = jnp.mean(y * y, axis=0, keepdims=True) - m * m
            gg = ga[...] * (y - m) * lax.rsqrt(v + eps) + be[...]
        res = jnp.dot(gg, wf[...], preferred_element_type=jnp.float32) + bfr[...]
        o_ref[...] = jnp.broadcast_to(res, (G, HH))

    cst = lambda: (0, 0)
    args = [g]
    in_specs = [pl.BlockSpec((G, H), cst)]
    for w, b, ga, be in zip(Ws, bs, gs, es):
        args += [w, b.reshape(1, H), ga.reshape(1, H), be.reshape(1, H)]
        in_specs += [pl.BlockSpec((H, H), cst), pl.BlockSpec((1, H), cst),
                     pl.BlockSpec((1, H), cst), pl.BlockSpec((1, H), cst)]
    args += [Wf, bf.reshape(1, 1)]
    in_specs += [pl.BlockSpec((H, 1), cst), pl.BlockSpec((1, 1), cst)]
    out = pl.pallas_call(
        body,
        in_specs=in_specs,
        out_specs=pl.BlockSpec((G, HH), cst),
        out_shape=jax.ShapeDtypeStruct((G, HH), jnp.float32),
    )(*args)
    return out[:, 0]


def kernel(x, edge_index_intra, edge_index_inter, pos, batch, params):
    row_i = edge_index_intra[0].astype(jnp.int32)
    col_i = edge_index_intra[1].astype(jnp.int32)
    row_n = edge_index_inter[0].astype(jnp.int32)
    col_n = edge_index_inter[1].astype(jnp.int32)

    posp = jnp.pad(pos.astype(jnp.float32), ((0, 0), (0, 125)))
    rowp = jnp.pad(jnp.concatenate([row_i, row_n]), (0, EPAD - 2 * E))
    colp = jnp.pad(jnp.concatenate([col_i, col_n]), (0, EPAD - 2 * E))
    d2E = _sc_d2(posp, rowp, colp).reshape(EPAD, 16)

    W, b = params["lin_node"]
    h2 = _lin0(x, W, b)

    batch3 = batch.astype(jnp.int32).reshape(NBLK, 1, BN)

    for p in params["hil"]:
        rcW = jnp.pad(p["rc_lin"][0], ((0, 7), (0, 0)))
        rnW = jnp.pad(p["rn_lin"][0], ((0, 7), (0, 0)))
        rad_i = _radial(d2E, rcW, p["rc_lin"][1], 0)
        rad_n = _radial(d2E, rnW, p["rn_lin"][1], NEBLK)
        oi2 = _edge_sc(h2, rad_i, row_i, col_i)
        on2 = _edge_sc(h2, rad_n, row_n, col_n)
        y1, y2, stats = _cov(h2, oi2, on2,
                             p["cov_lin"][0], p["cov_lin"][1],
                             p["ncov_lin"][0], p["ncov_lin"][1])
        h2 = _comb(y1, y2, stats,
                   p["cov_bn"][0], p["cov_bn"][1],
                   p["ncov_bn"][0], p["ncov_bn"][1])

    g = _seg(h2, batch3)
    return _head(g, params["fc"])


# speedup vs baseline: 1.8665x; 1.1706x over previous
import functools

import numpy as np

import jax
import jax.numpy as jnp
from jax import lax
from jax.experimental import pallas as pl
from jax.experimental.pallas import tpu as pltpu
from jax.experimental.pallas import tpu_sc as plsc

N = 10000
H = 256
HH = 128
E = 160000
G = 64
NC, NS = 2, 16
EPW = E // NS
EB = 80
NPS = 624
ZR = 128

EPAD = 327680
RB = 64
RCH = EPAD // (RB * NC * NS)


def _sc_d2(posp, rowp, colp):
    mesh = plsc.VectorSubcoreMesh(
        core_axis_name="c", subcore_axis_name="s",
        num_cores=NC, num_subcores=NS)

    @functools.partial(
        pl.kernel,
        out_type=jax.ShapeDtypeStruct((EPAD * 16,), jnp.float32),
        mesh=mesh,
        scratch_types=[
            pltpu.VMEM((RB,), jnp.int32), pltpu.VMEM((RB,), jnp.int32),
            pltpu.VMEM((RB,), jnp.int32), pltpu.VMEM((RB,), jnp.int32),
            pltpu.VMEM((RB, 128), jnp.float32), pltpu.VMEM((RB, 128), jnp.float32),
            pltpu.VMEM((RB, 128), jnp.float32), pltpu.VMEM((RB, 128), jnp.float32),
            pltpu.VMEM((RB * 16,), jnp.float32), pltpu.VMEM((RB * 16,), jnp.float32),
            pltpu.VMEM_SHARED((N, 128), jnp.float32),
            pltpu.SemaphoreType.DMA, pltpu.SemaphoreType.DMA,
            pltpu.SemaphoreType.DMA, pltpu.SemaphoreType.DMA,
        ],
    )
    def k(pos_hbm, row_hbm, col_hbm, out_hbm,
          rw0, rw1, cl0, cl1, pr0, pr1, pc0, pc1, ob0, ob1,
          ptab, si0, si1, sp0, sp1):
        c = lax.axis_index("c")
        s = lax.axis_index("s")
        wid = c * NS + s
        BUF = ((rw0, cl0, pr0, pc0, ob0, si0, sp0),
               (rw1, cl1, pr1, pc1, ob1, si1, sp1))

        pltpu.sync_copy(pos_hbm.at[pl.ds(s * NPS, 640)],
                        ptab.at[pl.ds(s * NPS, 640)])
        plsc.subcore_barrier()

        def base_of(g):
            return (wid * RCH + jnp.minimum(g, RCH - 1)) * RB

        def issue_in(g, b):
            rw, cl, pr, pc, ob, si, sp = BUF[b]
            base = base_of(g)
            pltpu.async_copy(row_hbm.at[pl.ds(base, RB)], rw, si)
            pltpu.async_copy(col_hbm.at[pl.ds(base, RB)], cl, si)

        def wait_in(b):
            rw, cl, pr, pc, ob, si, sp = BUF[b]
            pltpu.make_async_copy(row_hbm.at[pl.ds(0, RB)], rw, si).wait()
            pltpu.make_async_copy(col_hbm.at[pl.ds(0, RB)], cl, si).wait()

        def issue_gather(b):
            rw, cl, pr, pc, ob, si, sp = BUF[b]
            wait_in(b)
            pltpu.async_copy(ptab.at[rw], pr, sp)
            pltpu.async_copy(ptab.at[cl], pc, sp)

        def wait_gather(b):
            rw, cl, pr, pc, ob, si, sp = BUF[b]
            pltpu.make_async_copy(ptab.at[rw], pr, sp).wait()
            pltpu.make_async_copy(ptab.at[cl], pc, sp).wait()

        def compute(g, b):
            rw, cl, pr, pc, ob, si, sp = BUF[b]
            wait_gather(b)

            def edge(j, carry2):
                sl = pl.ds(0, 16)
                diff = pr[j, sl] - pc[j, sl]
                ob[pl.ds(j * 16, 16)] = diff * diff
                return carry2
            lax.fori_loop(0, RB, edge, 0, unroll=4)
            pltpu.sync_copy(ob, out_hbm.at[pl.ds(base_of(g) * 16, RB * 16)])

        issue_in(0, 0)
        issue_in(1, 1)
        issue_gather(0)

        def body(p, carry):
            g = 2 * p
            issue_gather(1)
            compute(g, 0)
            issue_in(g + 2, 0)
            issue_gather(0)
            compute(g + 1, 1)
            issue_in(g + 3, 1)
            return carry
        lax.fori_loop(0, RCH // 2, body, 0)
        wait_gather(0)
        wait_in(1)

    return k(posp, rowp, colp)


def _edge_sc(h2, rad2, row, col):
    mesh = plsc.VectorSubcoreMesh(
        core_axis_name="c", subcore_axis_name="s",
        num_cores=NC, num_subcores=NS)

    NCH = EPW // EB

    @functools.partial(
        pl.kernel,
        out_type=jax.ShapeDtypeStruct((NC * N, HH), jnp.float32),
        mesh=mesh,
        scratch_types=[
            pltpu.VMEM((EB,), jnp.int32), pltpu.VMEM((EB,), jnp.int32),
            pltpu.VMEM((EB,), jnp.int32), pltpu.VMEM((EB,), jnp.int32),
            pltpu.VMEM((EB,), jnp.int32), pltpu.VMEM((EB,), jnp.int32),
            pltpu.VMEM((EB, HH), jnp.float32), pltpu.VMEM((EB, HH), jnp.float32),
            pltpu.VMEM((EB, HH), jnp.float32), pltpu.VMEM((EB, HH), jnp.float32),
            pltpu.VMEM_SHARED((N, HH), jnp.float32),
            pltpu.SemaphoreType.DMA, pltpu.SemaphoreType.DMA,
            pltpu.SemaphoreType.DMA, pltpu.SemaphoreType.DMA,
            pltpu.SemaphoreType.DMA, pltpu.SemaphoreType.DMA,
        ],
    )
    def k(h_hbm, rad_hbm, row_hbm, col_hbm, out_hbm,
          rw0, rw1, cl0, cl1, gx0, gx1, hr0, hr1, rd0, rd1,
          acc, si0, si1, sr0, sr1, sg0, sg1):
        c = lax.axis_index("c")
        s = lax.axis_index("s")
        BUF = ((rw0, cl0, gx0, hr0, rd0, si0, sr0, sg0),
               (rw1, cl1, gx1, hr1, rd1, si1, sr1, sg1))
        zv = jnp.zeros((16,), jnp.float32)

        def zf(i, carry):
            r = i // (HH // 16)
            q = i % (HH // 16)
            hr0[r, pl.ds(q * 16, 16)] = zv
            return carry
        lax.fori_loop(0, EB * (HH // 16), zf, 0)
        for t in range(8):
            pltpu.sync_copy(hr0, acc.at[pl.ds(s * NPS + t * EB, EB)])
        plsc.subcore_barrier()

        cN = c * N

        def base_of(g):
            return s * EPW + jnp.minimum(g, NCH - 1) * EB

        def issue_in(g, b):
            rw, cl, gx, hr, rd, si, sr, sg = BUF[b]
            base = base_of(g)
            pltpu.async_copy(row_hbm.at[pl.ds(base, EB)], rw, si)
            pltpu.async_copy(col_hbm.at[pl.ds(base, EB)], cl, si)
            pltpu.async_copy(rad_hbm.at[pl.ds(c * E + base, EB)], rd, sr)

        def wait_in(b):
            rw, cl, gx, hr, rd, si, sr, sg = BUF[b]
            pltpu.make_async_copy(row_hbm.at[pl.ds(0, EB)], rw, si).wait()
            pltpu.make_async_copy(col_hbm.at[pl.ds(0, EB)], cl, si).wait()

        def issue_gather(b):
            rw, cl, gx, hr, rd, si, sr, sg = BUF[b]
            wait_in(b)
            for v in range(EB // 16):
                sl = pl.ds(v * 16, 16)
                gx[sl] = rw[sl] + cN
            pltpu.async_copy(h_hbm.at[gx], hr, sg)

        def mul_scatter(b):
            rw, cl, gx, hr, rd, si, sr, sg = BUF[b]
            pltpu.make_async_copy(h_hbm.at[gx], hr, sg).wait()
            pltpu.make_async_copy(rad_hbm.at[pl.ds(0, EB)], rd, sr).wait()

            def mrow(j, carry2):
                for v in range(HH // 16):
                    sl = pl.ds(v * 16, 16)
                    hr[j, sl] = hr[j, sl] * rd[j, sl]
                return carry2
            lax.fori_loop(0, EB, mrow, 0, unroll=2)
            pltpu.sync_copy(hr, acc.at[cl], add=True)

        issue_in(0, 0)
        issue_in(1, 1)
        issue_gather(0)

        def body(p, carry):
            g = 2 * p
            issue_gather(1)
            mul_scatter(0)
            issue_in(g + 2, 0)
            issue_gather(0)
            mul_scatter(1)
            issue_in(g + 3, 1)
            return carry
        lax.fori_loop(0, (NCH - 1) // 2, body, 0)
        mul_scatter(0)
        wait_in(1)
        rw, cl, gx, hr, rd, si, sr, sg = BUF[1]
        pltpu.make_async_copy(rad_hbm.at[pl.ds(0, EB)], rd, sr).wait()

        plsc.subcore_barrier()
        pltpu.sync_copy(acc.at[pl.ds(s * NPS, 640)],
                        out_hbm.at[pl.ds(cN + s * NPS, 640)])

    return k(h2, rad2, row, col)


BN = 1000
NBLK = N // BN
BE = 1280
NEBLK = E // BE


def _silu(x):
    return x * jax.nn.sigmoid(x)


def _lrelu(x):
    return jnp.where(x >= 0, x, 0.01 * x)


def _lin0(x, W, b):
    def body(x_ref, w_ref, b_ref, o_ref):
        y = _silu(jnp.dot(x_ref[...], w_ref[...],
                          preferred_element_type=jnp.float32) + b_ref[...])
        o_ref[0] = y[:, :HH]
        o_ref[1] = y[:, HH:]

    out = pl.pallas_call(
        body,
        grid=(NBLK,),
        in_specs=[
            pl.BlockSpec((BN, H), lambda i: (i, 0)),
            pl.BlockSpec((H, H), lambda i: (0, 0)),
            pl.BlockSpec((1, H), lambda i: (0, 0)),
        ],
        out_specs=pl.BlockSpec((2, BN, HH), lambda i: (0, i, 0)),
        out_shape=jax.ShapeDtypeStruct((2, N, HH), jnp.float32),
    )(x, W, b.reshape(1, H))
    return out.reshape(2 * N, HH)


def _radial(d2E, W, b, off):
    def body(r_ref, w_ref, b_ref, o_ref):
        sq = r_ref[...]
        d2 = sq[:, 0:1] + sq[:, 1:2] + sq[:, 2:3]
        d = jnp.sqrt(d2 + 1e-12)
        mu = lax.broadcasted_iota(jnp.int32, (BE, 16), 1).astype(jnp.float32) * 0.75
        t = (d - mu) * 1.5
        rbf = jnp.exp(-(t * t))
        y = jnp.dot(rbf, w_ref[...], preferred_element_type=jnp.float32)
        y = _silu(y + b_ref[...])
        o_ref[0] = y[:, :HH]
        o_ref[1] = y[:, HH:]

    out = pl.pallas_call(
        body,
        grid=(NEBLK,),
        in_specs=[
            pl.BlockSpec((BE, 16), lambda i: (i + off, 0)),
            pl.BlockSpec((16, H), lambda i: (0, 0)),
            pl.BlockSpec((1, H), lambda i: (0, 0)),
        ],
        out_specs=pl.BlockSpec((2, BE, HH), lambda i: (0, i, 0)),
        out_shape=jax.ShapeDtypeStruct((2, E, HH), jnp.float32),
    )(d2E, W, b.reshape(1, H))
    return out.reshape(2 * E, HH)


def _cov(h2, oi2, on2, W1, b1, W2, b2):
    def body(hl, hh, oil, oih, onl, onh, w1, bb1, w2, bb2,
             y1_ref, y2_ref, st_ref):
        i = pl.program_id(0)
        al = hl[...] + oil[...]
        ah = hh[...] + oih[...]
        y1 = _lrelu(jnp.dot(al, w1[:HH, :], preferred_element_type=jnp.float32)
                    + jnp.dot(ah, w1[HH:, :], preferred_element_type=jnp.float32)
                    + bb1[...])
        bl = hl[...] + onl[...]
        bh = hh[...] + onh[...]
        y2 = _lrelu(jnp.dot(bl, w2[:HH, :], preferred_element_type=jnp.float32)
                    + jnp.dot(bh, w2[HH:, :], preferred_element_type=jnp.float32)
                    + bb2[...])
        y1_ref[0] = y1[:, :HH]
        y1_ref[1] = y1[:, HH:]
        y2_ref[0] = y2[:, :HH]
        y2_ref[1] = y2[:, HH:]

        @pl.when(i == 0)
        def _():
            st_ref[...] = jnp.zeros_like(st_ref)
        st = jnp.concatenate([
            jnp.sum(y1, axis=0, keepdims=True),
            jnp.sum(y1 * y1, axis=0, keepdims=True),
            jnp.sum(y2, axis=0, keepdims=True),
            jnp.sum(y2 * y2, axis=0, keepdims=True),
        ], axis=0)
        st_ref[0:4, :] = st_ref[0:4, :] + st

    lo = lambda i: (i, 0)
    hi = lambda i: (i + NBLK, 0)
    cst = lambda i: (0, 0)
    y1, y2, stats = pl.pallas_call(
        body,
        grid=(NBLK,),
        in_specs=[
            pl.BlockSpec((BN, HH), lo), pl.BlockSpec((BN, HH), hi),
            pl.BlockSpec((BN, HH), lo), pl.BlockSpec((BN, HH), hi),
            pl.BlockSpec((BN, HH), lo), pl.BlockSpec((BN, HH), hi),
            pl.BlockSpec((H, H), cst), pl.BlockSpec((1, H), cst),
            pl.BlockSpec((H, H), cst), pl.BlockSpec((1, H), cst),
        ],
        out_specs=[
            pl.BlockSpec((2, BN, HH), lambda i: (0, i, 0)),
            pl.BlockSpec((2, BN, HH), lambda i: (0, i, 0)),
            pl.BlockSpec((8, H), cst),
        ],
        out_shape=[
            jax.ShapeDtypeStruct((2, N, HH), jnp.float32),
            jax.ShapeDtypeStruct((2, N, HH), jnp.float32),
            jax.ShapeDtypeStruct((8, H), jnp.float32),
        ],
    )(h2, h2, oi2, oi2, on2, on2, W1, b1.reshape(1, H), W2, b2.reshape(1, H))
    return y1.reshape(2 * N, HH), y2.reshape(2 * N, HH), stats


def _comb(y1, y2, stats, g1, be1, g2, be2, eps=1e-5):
    def body(y1l, y1h, y2l, y2h, st, gg1, bb1, gg2, bb2, o_ref):
        s = st[...]
        m1 = s[0:1, :] / N
        v1 = s[1:2, :] / N - m1 * m1
        sc1 = gg1[...] * lax.rsqrt(v1 + eps)
        sh1 = bb1[...] - m1 * sc1
        m2 = s[2:3, :] / N
        v2 = s[3:4, :] / N - m2 * m2
        sc2 = gg2[...] * lax.rsqrt(v2 + eps)
        sh2 = bb2[...] - m2 * sc2
        o_ref[0] = (y1l[...] * sc1[:, :HH] + sh1[:, :HH]
                    + y2l[...] * sc2[:, :HH] + sh2[:, :HH])
        o_ref[1] = (y1h[...] * sc1[:, HH:] + sh1[:, HH:]
                    + y2h[...] * sc2[:, HH:] + sh2[:, HH:])

    lo = lambda i: (i, 0)
    hi = lambda i: (i + NBLK, 0)
    cst = lambda i: (0, 0)
    out = pl.pallas_call(
        body,
        grid=(NBLK,),
        in_specs=[
            pl.BlockSpec((BN, HH), lo), pl.BlockSpec((BN, HH), hi),
            pl.BlockSpec((BN, HH), lo), pl.BlockSpec((BN, HH), hi),
            pl.BlockSpec((8, H), cst),
            pl.BlockSpec((1, H), cst), pl.BlockSpec((1, H), cst),
            pl.BlockSpec((1, H), cst), pl.BlockSpec((1, H), cst),
        ],
        out_specs=pl.BlockSpec((2, BN, HH), lambda i: (0, i, 0)),
        out_shape=jax.ShapeDtypeStruct((2, N, HH), jnp.float32),
    )(y1, y1, y2, y2, stats,
      g1.reshape(1, H), be1.reshape(1, H), g2.reshape(1, H), be2.reshape(1, H))
    return out.reshape(2 * N, HH)


def _seg(h2, batch3):
    def body(hl, hh, b_ref, g_ref):
        i = pl.program_id(0)

        @pl.when(i == 0)
        def _():
            g_ref[...] = jnp.zeros_like(g_ref)
        bb = b_ref[0]
        seg = lax.broadcasted_iota(jnp.int32, (G, BN), 0)
        oh = jnp.where(seg == jnp.broadcast_to(bb, (G, BN)), 1.0, 0.0)
        g_ref[:, :HH] = g_ref[:, :HH] + jnp.dot(
            oh, hl[...], preferred_element_type=jnp.float32)
        g_ref[:, HH:] = g_ref[:, HH:] + jnp.dot(
            oh, hh[...], preferred_element_type=jnp.float32)

    lo = lambda i: (i, 0)
    hi = lambda i: (i + NBLK, 0)
    return pl.pallas_call(
        body,
        grid=(NBLK,),
        in_specs=[
            pl.BlockSpec((BN, HH), lo), pl.BlockSpec((BN, HH), hi),
            pl.BlockSpec((1, 1, BN), lambda i: (i, 0, 0)),
        ],
        out_specs=pl.BlockSpec((G, H), lambda i: (0, 0)),
        out_shape=jax.ShapeDtypeStruct((G, H), jnp.float32),
    )(h2, h2, batch3)


def _head(g, fc, eps=1e-5):
    Ws = [w for w, _ in fc["lins"]]
    bs = [b for _, b in fc["lins"]]
    gs = [ga for ga, _ in fc["bns"]]
    es = [be for _, be in fc["bns"]]
    Wf, bf = fc["final"]

    def body(g_ref, w0, b0, g0, e0, w1, b1, g1, e1, w2, b2, g2, e2,
             wf, bfr, o_ref):
        gg = g_ref[...]
        for w, b, ga, be in ((w0, b0, g0, e0), (w1, b1, g1, e1),
                             (w2, b2, g2, e2)):
            y = _lrelu(jnp.dot(gg, w[...],
                               preferred_element_type=jnp.float32) + b[...])
            m = jnp.mean(y, axis=0, keepdims=True)
            v = jnp.mean(y * y, axis=0, keepdims=True) - m * m
            gg = ga[...] * (y - m) * lax.rsqrt(v + eps) + be[...]
        res = jnp.dot(gg, wf[...], preferred_element_type=jnp.float32) + bfr[...]
        o_ref[...] = jnp.broadcast_to(res, (G, HH))

    cst = lambda: (0, 0)
    args = [g]
    in_specs = [pl.BlockSpec((G, H), cst)]
    for w, b, ga, be in zip(Ws, bs, gs, es):
        args += [w, b.reshape(1, H), ga.reshape(1, H), be.reshape(1, H)]
        in_specs += [pl.BlockSpec((H, H), cst), pl.BlockSpec((1, H), cst),
                     pl.BlockSpec((1, H), cst), pl.BlockSpec((1, H), cst)]
    args += [Wf, bf.reshape(1, 1)]
    in_specs += [pl.BlockSpec((H, 1), cst), pl.BlockSpec((1, 1), cst)]
    out = pl.pallas_call(
        body,
        in_specs=in_specs,
        out_specs=pl.BlockSpec((G, HH), cst),
        out_shape=jax.ShapeDtypeStruct((G, HH), jnp.float32),
    )(*args)
    return out[:, 0]


def kernel(x, edge_index_intra, edge_index_inter, pos, batch, params):
    row_i = edge_index_intra[0].astype(jnp.int32)
    col_i = edge_index_intra[1].astype(jnp.int32)
    row_n = edge_index_inter[0].astype(jnp.int32)
    col_n = edge_index_inter[1].astype(jnp.int32)

    posp = jnp.pad(pos.astype(jnp.float32), ((0, 0), (0, 125)))
    rowp = jnp.pad(jnp.concatenate([row_i, row_n]), (0, EPAD - 2 * E))
    colp = jnp.pad(jnp.concatenate([col_i, col_n]), (0, EPAD - 2 * E))
    d2E = _sc_d2(posp, rowp, colp).reshape(EPAD, 16)

    W, b = params["lin_node"]
    h2 = _lin0(x, W, b)

    batch3 = batch.astype(jnp.int32).reshape(NBLK, 1, BN)

    for p in params["hil"]:
        rcW = jnp.pad(p["rc_lin"][0], ((0, 7), (0, 0)))
        rnW = jnp.pad(p["rn_lin"][0], ((0, 7), (0, 0)))
        rad_i = _radial(d2E, rcW, p["rc_lin"][1], 0)
        rad_n = _radial(d2E, rnW, p["rn_lin"][1], NEBLK)
        oi2 = _edge_sc(h2, rad_i, row_i, col_i)
        on2 = _edge_sc(h2, rad_n, row_n, col_n)
        y1, y2, stats = _cov(h2, oi2, on2,
                             p["cov_lin"][0], p["cov_lin"][1],
                             p["ncov_lin"][0], p["ncov_lin"][1])
        h2 = _comb(y1, y2, stats,
                   p["cov_bn"][0], p["cov_bn"][1],
                   p["ncov_bn"][0], p["ncov_bn"][1])

    g = _seg(h2, batch3)
    return _head(g, params["fc"])


# speedup vs baseline: 1.8700x; 1.0019x over previous
import functools

import numpy as np

import jax
import jax.numpy as jnp
from jax import lax
from jax.experimental import pallas as pl
from jax.experimental.pallas import tpu as pltpu
from jax.experimental.pallas import tpu_sc as plsc

N = 10000
H = 256
HH = 128
E = 160000
G = 64
NC, NS = 2, 16
EPW = E // NS
EB = 80
NPS = 624
ZR = 128

EPAD = 327680
RB = 64
RCH = EPAD // (RB * NC * NS)


def _sc_d2(posp, rowp, colp):
    mesh = plsc.VectorSubcoreMesh(
        core_axis_name="c", subcore_axis_name="s",
        num_cores=NC, num_subcores=NS)

    @functools.partial(
        pl.kernel,
        out_type=jax.ShapeDtypeStruct((EPAD * 16,), jnp.float32),
        mesh=mesh,
        scratch_types=[
            pltpu.VMEM((RB,), jnp.int32), pltpu.VMEM((RB,), jnp.int32),
            pltpu.VMEM((RB,), jnp.int32), pltpu.VMEM((RB,), jnp.int32),
            pltpu.VMEM((RB, 128), jnp.float32), pltpu.VMEM((RB, 128), jnp.float32),
            pltpu.VMEM((RB, 128), jnp.float32), pltpu.VMEM((RB, 128), jnp.float32),
            pltpu.VMEM((RB * 16,), jnp.float32), pltpu.VMEM((RB * 16,), jnp.float32),
            pltpu.VMEM_SHARED((N, 128), jnp.float32),
            pltpu.SemaphoreType.DMA, pltpu.SemaphoreType.DMA,
            pltpu.SemaphoreType.DMA, pltpu.SemaphoreType.DMA,
        ],
    )
    def k(pos_hbm, row_hbm, col_hbm, out_hbm,
          rw0, rw1, cl0, cl1, pr0, pr1, pc0, pc1, ob0, ob1,
          ptab, si0, si1, sp0, sp1):
        c = lax.axis_index("c")
        s = lax.axis_index("s")
        wid = c * NS + s
        BUF = ((rw0, cl0, pr0, pc0, ob0, si0, sp0),
               (rw1, cl1, pr1, pc1, ob1, si1, sp1))

        pltpu.sync_copy(pos_hbm.at[pl.ds(s * NPS, 640)],
                        ptab.at[pl.ds(s * NPS, 640)])
        plsc.subcore_barrier()

        def base_of(g):
            return (wid * RCH + jnp.minimum(g, RCH - 1)) * RB

        def issue_in(g, b):
            rw, cl, pr, pc, ob, si, sp = BUF[b]
            base = base_of(g)
            pltpu.async_copy(row_hbm.at[pl.ds(base, RB)], rw, si)
            pltpu.async_copy(col_hbm.at[pl.ds(base, RB)], cl, si)

        def wait_in(b):
            rw, cl, pr, pc, ob, si, sp = BUF[b]
            pltpu.make_async_copy(row_hbm.at[pl.ds(0, RB)], rw, si).wait()
            pltpu.make_async_copy(col_hbm.at[pl.ds(0, RB)], cl, si).wait()

        def issue_gather(b):
            rw, cl, pr, pc, ob, si, sp = BUF[b]
            wait_in(b)
            pltpu.async_copy(ptab.at[rw], pr, sp)
            pltpu.async_copy(ptab.at[cl], pc, sp)

        def wait_gather(b):
            rw, cl, pr, pc, ob, si, sp = BUF[b]
            pltpu.make_async_copy(ptab.at[rw], pr, sp).wait()
            pltpu.make_async_copy(ptab.at[cl], pc, sp).wait()

        def compute(g, b):
            rw, cl, pr, pc, ob, si, sp = BUF[b]
            wait_gather(b)

            def edge(j, carry2):
                sl = pl.ds(0, 16)
                diff = pr[j, sl] - pc[j, sl]
                ob[pl.ds(j * 16, 16)] = diff * diff
                return carry2
            lax.fori_loop(0, RB, edge, 0, unroll=4)
            pltpu.sync_copy(ob, out_hbm.at[pl.ds(base_of(g) * 16, RB * 16)])

        issue_in(0, 0)
        issue_in(1, 1)
        issue_gather(0)

        def body(p, carry):
            g = 2 * p
            issue_gather(1)
            compute(g, 0)
            issue_in(g + 2, 0)
            issue_gather(0)
            compute(g + 1, 1)
            issue_in(g + 3, 1)
            return carry
        lax.fori_loop(0, RCH // 2, body, 0)
        wait_gather(0)
        wait_in(1)

    return k(posp, rowp, colp)


def _edge_sc(h2, radi2, radn2, row_i, col_i, row_n, col_n):
    mesh = plsc.VectorSubcoreMesh(
        core_axis_name="c", subcore_axis_name="s",
        num_cores=NC, num_subcores=NS)

    NCH = EPW // EB

    @functools.partial(
        pl.kernel,
        out_type=[jax.ShapeDtypeStruct((NC * N, HH), jnp.float32),
                  jax.ShapeDtypeStruct((NC * N, HH), jnp.float32)],
        mesh=mesh,
        scratch_types=[
            pltpu.VMEM((EB,), jnp.int32), pltpu.VMEM((EB,), jnp.int32),
            pltpu.VMEM((EB,), jnp.int32), pltpu.VMEM((EB,), jnp.int32),
            pltpu.VMEM((EB,), jnp.int32), pltpu.VMEM((EB,), jnp.int32),
            pltpu.VMEM((EB, HH), jnp.float32), pltpu.VMEM((EB, HH), jnp.float32),
            pltpu.VMEM((EB, HH), jnp.float32), pltpu.VMEM((EB, HH), jnp.float32),
            pltpu.VMEM_SHARED((N, HH), jnp.float32),
            pltpu.SemaphoreType.DMA, pltpu.SemaphoreType.DMA,
            pltpu.SemaphoreType.DMA, pltpu.SemaphoreType.DMA,
            pltpu.SemaphoreType.DMA, pltpu.SemaphoreType.DMA,
        ],
    )
    def k(h_hbm, radi_hbm, radn_hbm, rowi_hbm, coli_hbm, rown_hbm, coln_hbm,
          outi_hbm, outn_hbm,
          rw0, rw1, cl0, cl1, gx0, gx1, hr0, hr1, rd0, rd1,
          acc, si0, si1, sr0, sr1, sg0, sg1):
        c = lax.axis_index("c")
        s = lax.axis_index("s")
        BUF = ((rw0, cl0, gx0, hr0, rd0, si0, sr0, sg0),
               (rw1, cl1, gx1, hr1, rd1, si1, sr1, sg1))
        zv = jnp.zeros((16,), jnp.float32)
        cN = c * N

        for (rad_hbm, row_hbm, col_hbm, out_hbm) in (
                (radi_hbm, rowi_hbm, coli_hbm, outi_hbm),
                (radn_hbm, rown_hbm, coln_hbm, outn_hbm)):

            def zf(i, carry):
                r = i // (HH // 16)
                q = i % (HH // 16)
                hr0[r, pl.ds(q * 16, 16)] = zv
                return carry
            lax.fori_loop(0, EB * (HH // 16), zf, 0)
            for t in range(8):
                pltpu.sync_copy(hr0, acc.at[pl.ds(s * NPS + t * EB, EB)])
            plsc.subcore_barrier()

            def base_of(g):
                return s * EPW + jnp.minimum(g, NCH - 1) * EB

            def issue_in(g, b):
                rw, cl, gx, hr, rd, si, sr, sg = BUF[b]
                base = base_of(g)
                pltpu.async_copy(row_hbm.at[pl.ds(base, EB)], rw, si)
                pltpu.async_copy(col_hbm.at[pl.ds(base, EB)], cl, si)
                pltpu.async_copy(rad_hbm.at[pl.ds(c * E + base, EB)], rd, sr)

            def wait_in(b):
                rw, cl, gx, hr, rd, si, sr, sg = BUF[b]
                pltpu.make_async_copy(row_hbm.at[pl.ds(0, EB)], rw, si).wait()
                pltpu.make_async_copy(col_hbm.at[pl.ds(0, EB)], cl, si).wait()

            def issue_gather(b):
                rw, cl, gx, hr, rd, si, sr, sg = BUF[b]
                wait_in(b)
                for v in range(EB // 16):
                    sl = pl.ds(v * 16, 16)
                    gx[sl] = rw[sl] + cN
                pltpu.async_copy(h_hbm.at[gx], hr, sg)

            def mul_scatter(b):
                rw, cl, gx, hr, rd, si, sr, sg = BUF[b]
                pltpu.make_async_copy(h_hbm.at[gx], hr, sg).wait()
                pltpu.make_async_copy(rad_hbm.at[pl.ds(0, EB)], rd, sr).wait()

                def mrow(j, carry2):
                    for v in range(HH // 16):
                        sl = pl.ds(v * 16, 16)
                        hr[j, sl] = hr[j, sl] * rd[j, sl]
                    return carry2
                lax.fori_loop(0, EB, mrow, 0, unroll=4)
                pltpu.sync_copy(hr, acc.at[cl], add=True)

            issue_in(0, 0)
            issue_in(1, 1)
            issue_gather(0)

            def body(p, carry):
                g = 2 * p
                issue_gather(1)
                mul_scatter(0)
                issue_in(g + 2, 0)
                issue_gather(0)
                mul_scatter(1)
                issue_in(g + 3, 1)
                return carry
            lax.fori_loop(0, (NCH - 1) // 2, body, 0)
            mul_scatter(0)
            wait_in(1)
            rwd, cld, gxd, hrd, rdd, sid, srd, sgd = BUF[1]
            pltpu.make_async_copy(rad_hbm.at[pl.ds(0, EB)], rdd, srd).wait()

            plsc.subcore_barrier()
            pltpu.sync_copy(acc.at[pl.ds(s * NPS, 640)],
                            out_hbm.at[pl.ds(cN + s * NPS, 640)])
            plsc.subcore_barrier()

    return k(h2, radi2, radn2, row_i, col_i, row_n, col_n)


BN = 1000
NBLK = N // BN
BE = 1280
NEBLK = E // BE


def _silu(x):
    return x * jax.nn.sigmoid(x)


def _lrelu(x):
    return jnp.where(x >= 0, x, 0.01 * x)


def _lin0(x, W, b):
    def body(x_ref, w_ref, b_ref, o_ref):
        y = _silu(jnp.dot(x_ref[...], w_ref[...],
                          preferred_element_type=jnp.float32) + b_ref[...])
        o_ref[0] = y[:, :HH]
        o_ref[1] = y[:, HH:]

    out = pl.pallas_call(
        body,
        grid=(NBLK,),
        in_specs=[
            pl.BlockSpec((BN, H), lambda i: (i, 0)),
            pl.BlockSpec((H, H), lambda i: (0, 0)),
            pl.BlockSpec((1, H), lambda i: (0, 0)),
        ],
        out_specs=pl.BlockSpec((2, BN, HH), lambda i: (0, i, 0)),
        out_shape=jax.ShapeDtypeStruct((2, N, HH), jnp.float32),
    )(x, W, b.reshape(1, H))
    return out.reshape(2 * N, HH)


def _radial(d2E, W, b, off):
    def body(r_ref, w_ref, b_ref, o_ref):
        sq = r_ref[...]
        d2 = sq[:, 0:1] + sq[:, 1:2] + sq[:, 2:3]
        d = jnp.sqrt(d2 + 1e-12)
        mu = lax.broadcasted_iota(jnp.int32, (BE, 16), 1).astype(jnp.float32) * 0.75
        t = (d - mu) * 1.5
        rbf = jnp.exp(-(t * t))
        y = jnp.dot(rbf, w_ref[...], preferred_element_type=jnp.float32)
        y = _silu(y + b_ref[...])
        o_ref[0] = y[:, :HH]
        o_ref[1] = y[:, HH:]

    out = pl.pallas_call(
        body,
        grid=(NEBLK,),
        in_specs=[
            pl.BlockSpec((BE, 16), lambda i: (i + off, 0)),
            pl.BlockSpec((16, H), lambda i: (0, 0)),
            pl.BlockSpec((1, H), lambda i: (0, 0)),
        ],
        out_specs=pl.BlockSpec((2, BE, HH), lambda i: (0, i, 0)),
        out_shape=jax.ShapeDtypeStruct((2, E, HH), jnp.float32),
    )(d2E, W, b.reshape(1, H))
    return out.reshape(2 * E, HH)


def _cov(h2, oi2, on2, W1, b1, W2, b2):
    def body(hl, hh, oil, oih, onl, onh, w1, bb1, w2, bb2,
             y1_ref, y2_ref, st_ref):
        i = pl.program_id(0)
        al = hl[...] + oil[...]
        ah = hh[...] + oih[...]
        y1 = _lrelu(jnp.dot(al, w1[:HH, :], preferred_element_type=jnp.float32)
                    + jnp.dot(ah, w1[HH:, :], preferred_element_type=jnp.float32)
                    + bb1[...])
        bl = hl[...] + onl[...]
        bh = hh[...] + onh[...]
        y2 = _lrelu(jnp.dot(bl, w2[:HH, :], preferred_element_type=jnp.float32)
                    + jnp.dot(bh, w2[HH:, :], preferred_element_type=jnp.float32)
                    + bb2[...])
        y1_ref[0] = y1[:, :HH]
        y1_ref[1] = y1[:, HH:]
        y2_ref[0] = y2[:, :HH]
        y2_ref[1] = y2[:, HH:]

        @pl.when(i == 0)
        def _():
            st_ref[...] = jnp.zeros_like(st_ref)
        st = jnp.concatenate([
            jnp.sum(y1, axis=0, keepdims=True),
            jnp.sum(y1 * y1, axis=0, keepdims=True),
            jnp.sum(y2, axis=0, keepdims=True),
            jnp.sum(y2 * y2, axis=0, keepdims=True),
        ], axis=0)
        st_ref[0:4, :] = st_ref[0:4, :] + st

    lo = lambda i: (i, 0)
    hi = lambda i: (i + NBLK, 0)
    cst = lambda i: (0, 0)
    y1, y2, stats = pl.pallas_call(
        body,
        grid=(NBLK,),
        in_specs=[
            pl.BlockSpec((BN, HH), lo), pl.BlockSpec((BN, HH), hi),
            pl.BlockSpec((BN, HH), lo), pl.BlockSpec((BN, HH), hi),
            pl.BlockSpec((BN, HH), lo), pl.BlockSpec((BN, HH), hi),
            pl.BlockSpec((H, H), cst), pl.BlockSpec((1, H), cst),
            pl.BlockSpec((H, H), cst), pl.BlockSpec((1, H), cst),
        ],
        out_specs=[
            pl.BlockSpec((2, BN, HH), lambda i: (0, i, 0)),
            pl.BlockSpec((2, BN, HH), lambda i: (0, i, 0)),
            pl.BlockSpec((8, H), cst),
        ],
        out_shape=[
            jax.ShapeDtypeStruct((2, N, HH), jnp.float32),
            jax.ShapeDtypeStruct((2, N, HH), jnp.float32),
            jax.ShapeDtypeStruct((8, H), jnp.float32),
        ],
    )(h2, h2, oi2, oi2, on2, on2, W1, b1.reshape(1, H), W2, b2.reshape(1, H))
    return y1.reshape(2 * N, HH), y2.reshape(2 * N, HH), stats


def _comb(y1, y2, stats, g1, be1, g2, be2, eps=1e-5):
    def body(y1l, y1h, y2l, y2h, st, gg1, bb1, gg2, bb2, o_ref):
        s = st[...]
        m1 = s[0:1, :] / N
        v1 = s[1:2, :] / N - m1 * m1
        sc1 = gg1[...] * lax.rsqrt(v1 + eps)
        sh1 = bb1[...] - m1 * sc1
        m2 = s[2:3, :] / N
        v2 = s[3:4, :] / N - m2 * m2
        sc2 = gg2[...] * lax.rsqrt(v2 + eps)
        sh2 = bb2[...] - m2 * sc2
        o_ref[0] = (y1l[...] * sc1[:, :HH] + sh1[:, :HH]
                    + y2l[...] * sc2[:, :HH] + sh2[:, :HH])
        o_ref[1] = (y1h[...] * sc1[:, HH:] + sh1[:, HH:]
                    + y2h[...] * sc2[:, HH:] + sh2[:, HH:])

    lo = lambda i: (i, 0)
    hi = lambda i: (i + NBLK, 0)
    cst = lambda i: (0, 0)
    out = pl.pallas_call(
        body,
        grid=(NBLK,),
        in_specs=[
            pl.BlockSpec((BN, HH), lo), pl.BlockSpec((BN, HH), hi),
            pl.BlockSpec((BN, HH), lo), pl.BlockSpec((BN, HH), hi),
            pl.BlockSpec((8, H), cst),
            pl.BlockSpec((1, H), cst), pl.BlockSpec((1, H), cst),
            pl.BlockSpec((1, H), cst), pl.BlockSpec((1, H), cst),
        ],
        out_specs=pl.BlockSpec((2, BN, HH), lambda i: (0, i, 0)),
        out_shape=jax.ShapeDtypeStruct((2, N, HH), jnp.float32),
    )(y1, y1, y2, y2, stats,
      g1.reshape(1, H), be1.reshape(1, H), g2.reshape(1, H), be2.reshape(1, H))
    return out.reshape(2 * N, HH)


def _seg(h2, batch3):
    def body(hl, hh, b_ref, g_ref):
        i = pl.program_id(0)

        @pl.when(i == 0)
        def _():
            g_ref[...] = jnp.zeros_like(g_ref)
        bb = b_ref[0]
        seg = lax.broadcasted_iota(jnp.int32, (G, BN), 0)
        oh = jnp.where(seg == jnp.broadcast_to(bb, (G, BN)), 1.0, 0.0)
        g_ref[:, :HH] = g_ref[:, :HH] + jnp.dot(
            oh, hl[...], preferred_element_type=jnp.float32)
        g_ref[:, HH:] = g_ref[:, HH:] + jnp.dot(
            oh, hh[...], preferred_element_type=jnp.float32)

    lo = lambda i: (i, 0)
    hi = lambda i: (i + NBLK, 0)
    return pl.pallas_call(
        body,
        grid=(NBLK,),
        in_specs=[
            pl.BlockSpec((BN, HH), lo), pl.BlockSpec((BN, HH), hi),
            pl.BlockSpec((1, 1, BN), lambda i: (i, 0, 0)),
        ],
        out_specs=pl.BlockSpec((G, H), lambda i: (0, 0)),
        out_shape=jax.ShapeDtypeStruct((G, H), jnp.float32),
    )(h2, h2, batch3)


def _head(g, fc, eps=1e-5):
    Ws = [w for w, _ in fc["lins"]]
    bs = [b for _, b in fc["lins"]]
    gs = [ga for ga, _ in fc["bns"]]
    es = [be for _, be in fc["bns"]]
    Wf, bf = fc["final"]

    def body(g_ref, w0, b0, g0, e0, w1, b1, g1, e1, w2, b2, g2, e2,
             wf, bfr, o_ref):
        gg = g_ref[...]
        for w, b, ga, be in ((w0, b0, g0, e0), (w1, b1, g1, e1),
                             (w2, b2, g2, e2)):
            y = _lrelu(jnp.dot(gg, w[...],
                               preferred_element_type=jnp.float32) + b[...])
            m = jnp.mean(y, axis=0, keepdims=True)
            v = jnp.mean(y * y, axis=0, keepdims=True) - m * m
            gg = ga[...] * (y - m) * lax.rsqrt(v + eps) + be[...]
        res = jnp.dot(gg, wf[...], preferred_element_type=jnp.float32) + bfr[...]
        o_ref[...] = jnp.broadcast_to(res, (G, HH))

    cst = lambda: (0, 0)
    args = [g]
    in_specs = [pl.BlockSpec((G, H), cst)]
    for w, b, ga, be in zip(Ws, bs, gs, es):
        args += [w, b.reshape(1, H), ga.reshape(1, H), be.reshape(1, H)]
        in_specs += [pl.BlockSpec((H, H), cst), pl.BlockSpec((1, H), cst),
                     pl.BlockSpec((1, H), cst), pl.BlockSpec((1, H), cst)]
    args += [Wf, bf.reshape(1, 1)]
    in_specs += [pl.BlockSpec((H, 1), cst), pl.BlockSpec((1, 1), cst)]
    out = pl.pallas_call(
        body,
        in_specs=in_specs,
        out_specs=pl.BlockSpec((G, HH), cst),
        out_shape=jax.ShapeDtypeStruct((G, HH), jnp.float32),
    )(*args)
    return out[:, 0]


def kernel(x, edge_index_intra, edge_index_inter, pos, batch, params):
    row_i = edge_index_intra[0].astype(jnp.int32)
    col_i = edge_index_intra[1].astype(jnp.int32)
    row_n = edge_index_inter[0].astype(jnp.int32)
    col_n = edge_index_inter[1].astype(jnp.int32)

    posp = jnp.pad(pos.astype(jnp.float32), ((0, 0), (0, 125)))
    rowp = jnp.pad(jnp.concatenate([row_i, row_n]), (0, EPAD - 2 * E))
    colp = jnp.pad(jnp.concatenate([col_i, col_n]), (0, EPAD - 2 * E))
    d2E = _sc_d2(posp, rowp, colp).reshape(EPAD, 16)

    W, b = params["lin_node"]
    h2 = _lin0(x, W, b)

    batch3 = batch.astype(jnp.int32).reshape(NBLK, 1, BN)

    for p in params["hil"]:
        rcW = jnp.pad(p["rc_lin"][0], ((0, 7), (0, 0)))
        rnW = jnp.pad(p["rn_lin"][0], ((0, 7), (0, 0)))
        rad_i = _radial(d2E, rcW, p["rc_lin"][1], 0)
        rad_n = _radial(d2E, rnW, p["rn_lin"][1], NEBLK)
        oi2, on2 = _edge_sc(h2, rad_i, rad_n, row_i, col_i, row_n, col_n)
        y1, y2, stats = _cov(h2, oi2, on2,
                             p["cov_lin"][0], p["cov_lin"][1],
                             p["ncov_lin"][0], p["ncov_lin"][1])
        h2 = _comb(y1, y2, stats,
                   p["cov_bn"][0], p["cov_bn"][1],
                   p["ncov_bn"][0], p["ncov_bn"][1])

    g = _seg(h2, batch3)
    return _head(g, params["fc"])
